# Initial kernel scaffold; baseline (speedup 1.0000x reference)
import jax
import jax.numpy as jnp
from jax.experimental import pallas as pl


def kernel(fe, gemm_edges, W, b):
    raise NotImplementedError("write your pallas kernel here")


# speedup vs baseline: 5.0944x; 5.0944x over previous
import functools

import jax
import jax.numpy as jnp
from jax import lax
from jax.experimental import pallas as pl
from jax.experimental.pallas import tpu as pltpu
from jax.experimental.pallas import tpu_sc as plsc

EPS = 1e-5

_NC = 2
_NS = 16
_NW = _NC * _NS

_CB = 40


def _sc_gather(xt, idx3):
    E, C = xt.shape
    EW = E // _NW
    NCH = EW // _CB

    mesh = plsc.VectorSubcoreMesh(core_axis_name="c", subcore_axis_name="s")

    @functools.partial(
        pl.kernel,
        out_type=jax.ShapeDtypeStruct((4, E, C), jnp.float32),
        mesh=mesh,
        scratch_types=[
            [pltpu.VMEM((EW,), jnp.int32) for _ in range(4)],
            pltpu.VMEM((4, _CB, C), jnp.float32),
            pltpu.VMEM((4, _CB, C), jnp.float32),
            pltpu.SemaphoreType.DMA,
            pltpu.SemaphoreType.DMA,
        ],
    )
    def gather_kernel(xt_hbm, idx3_hbm, g_hbm, idx_v, bufa, bufb, sema, semb):
        wid = lax.axis_index("s") * _NC + lax.axis_index("c")
        base = wid * EW
        for k in range(4):
            pltpu.sync_copy(idx3_hbm.at[wid, k], idx_v[k])

        def start(chunk, buf, sem):
            off = chunk * _CB
            for k in range(4):
                pltpu.async_copy(
                    xt_hbm.at[idx_v[k].at[pl.ds(off, _CB)]], buf.at[k], sem)

        def wait4(buf, sem):
            for k in range(4):
                pltpu.make_async_copy(
                    xt_hbm.at[pl.ds(0, _CB)], buf.at[k], sem).wait()

        def write(chunk, buf):
            off = base + chunk * _CB
            for k in range(4):
                pltpu.sync_copy(buf.at[k], g_hbm.at[k, pl.ds(off, _CB)])

        start(0, bufa, sema)

        def body(t, carry):
            i = t * 2
            wait4(bufa, sema)
            start(i + 1, bufb, semb)
            write(i, bufa)
            wait4(bufb, semb)
            start(i + 2, bufa, sema)
            write(i + 1, bufb)
            return carry

        lax.fori_loop(0, (NCH - 1) // 2, body, 0)
        wait4(bufa, sema)
        write(NCH - 1, bufa)

    return gather_kernel(xt, idx3)


def _tc_conv(fe, g, wt, b2):
    _, C, E = fe.shape
    c_out = wt.shape[1]
    EB = 1280
    nb = E // EB

    def body(fe_ref, g_ref, w_ref, b_ref, y_ref, st_ref, sacc, qacc):
        i = pl.program_id(0)

        @pl.when(i == 0)
        def _():
            sacc[...] = jnp.zeros_like(sacc)
            qacc[...] = jnp.zeros_like(qacc)

        xb = fe_ref[0]
        w = w_ref[...]
        gg = g_ref[...]
        s13 = gg[0] + gg[2]
        s24 = gg[1] + gg[3]
        d13 = jnp.abs(gg[0] - gg[2])
        d24 = jnp.abs(gg[1] - gg[3])
        dn = (((1,), (1,)), ((), ()))
        acc = jnp.dot(w[0], xb, preferred_element_type=jnp.float32)
        acc += lax.dot_general(w[1], s13, dn, preferred_element_type=jnp.float32)
        acc += lax.dot_general(w[2], s24, dn, preferred_element_type=jnp.float32)
        acc += lax.dot_general(w[3], d13, dn, preferred_element_type=jnp.float32)
        acc += lax.dot_general(w[4], d24, dn, preferred_element_type=jnp.float32)
        y = jnp.maximum(acc + b_ref[...], 0.0)
        y_ref[...] = y
        sacc[...] += jnp.sum(y, axis=1, keepdims=True)
        qacc[...] += jnp.sum(y * y, axis=1, keepdims=True)

        @pl.when(i == nb - 1)
        def _():
            st_ref[...] = jnp.concatenate([sacc[...], qacc[...]], axis=1)

    return pl.pallas_call(
        body,
        grid=(nb,),
        in_specs=[
            pl.BlockSpec((1, C, EB), lambda i: (0, 0, i)),
            pl.BlockSpec((4, EB, C), lambda i: (0, i, 0)),
            pl.BlockSpec((5, c_out, C), lambda i: (0, 0, 0)),
            pl.BlockSpec((c_out, 1), lambda i: (0, 0)),
        ],
        out_specs=[
            pl.BlockSpec((c_out, EB), lambda i: (0, i)),
            pl.BlockSpec((c_out, 2), lambda i: (0, 0)),
        ],
        out_shape=[
            jax.ShapeDtypeStruct((c_out, E), jnp.float32),
            jax.ShapeDtypeStruct((c_out, 2), jnp.float32),
        ],
        scratch_shapes=[
            pltpu.VMEM((c_out, 1), jnp.float32),
            pltpu.VMEM((c_out, 1), jnp.float32),
        ],
        compiler_params=pltpu.CompilerParams(
            dimension_semantics=("arbitrary",)),
    )(fe, g, wt, b2)


def _tc_norm(y, st):
    c_out, E = y.shape
    EB = 1280
    nb = E // EB
    inv_e = float(1.0 / E)

    def body(y_ref, st_ref, o_ref):
        stv = st_ref[...]
        mu = stv[:, 0:1] * inv_e
        var = stv[:, 1:2] * inv_e - mu * mu
        r = lax.rsqrt(var + EPS)
        o_ref[...] = ((y_ref[...] - mu) * r)[None]

    return pl.pallas_call(
        body,
        grid=(nb,),
        in_specs=[
            pl.BlockSpec((c_out, EB), lambda i: (0, i)),
            pl.BlockSpec((c_out, 2), lambda i: (0, 0)),
        ],
        out_specs=pl.BlockSpec((1, c_out, EB), lambda i: (0, 0, i)),
        out_shape=jax.ShapeDtypeStruct((1, c_out, E), jnp.float32),
        compiler_params=pltpu.CompilerParams(
            dimension_semantics=("arbitrary",)),
    )(y, st)


def kernel(fe, gemm_edges, W, b):
    _, C, E = fe.shape
    c_out = W.shape[0]
    xt = jnp.transpose(fe[0])
    ew = E // _NW
    idx3 = jnp.transpose(gemm_edges[0].reshape(_NW, ew, 4), (0, 2, 1))
    g = _sc_gather(xt, idx3)
    wt = jnp.transpose(W, (2, 0, 1))
    b2 = b.reshape(c_out, 1)
    y, st = _tc_conv(fe, g, wt, b2)
    return _tc_norm(y, st)
